# baseline (speedup 1.0000x reference)
import jax
import jax.numpy as jnp
from jax import lax
from jax.experimental import pallas as pl
from jax.experimental.pallas import tpu as pltpu
from jax.experimental.pallas import tpu_sc as plsc

IVS = 100000
T = 16384
H = 128
U = 512
S = 8
NPOOL = 64
PLEN = 2048
NCELL = NPOOL * PLEN
NC, NS = 2, 16
NW = NC * NS
TPW = T // NW
UPS = U // NS
L = 16


def _iota():
    return lax.iota(jnp.int32, L)


def _bcast(x):
    return jnp.broadcast_to(x, (L,)).astype(jnp.int32)


def _splat_ref(ref, idx):
    return plsc.load_gather(ref, [_bcast(idx)])


def _body(input_hbm, embed_hbm, pool_hbm, upd_hbm, uidx_hbm, utb_hbm,
          hwpi_hbm, ttb_hbm, out_hbm,
          ids_v, ttb_v, hwpi_v, uids_v, uidx_v, utb_v, cells_all_v,
          rows_v, urows_a, encbuf_v, row0_v, recip_v, npadf_v,
          initbuf_v, scidx_v, scval_v, bidx_v, cells_v, vvals_v,
          bufpos_v, encpos_v, hrows_v, outidx_v,
          enc_s, ver_s, sem, sem2, sem3, semu):
    c = lax.axis_index("c")
    s = lax.axis_index("s")
    wid = c * NS + s
    t0 = wid * TPW
    us0 = s * UPS

    stg = [
        pltpu.async_copy(uidx_hbm, uidx_v, sem),
        pltpu.async_copy(utb_hbm, utb_v, sem),
        pltpu.async_copy(hwpi_hbm, hwpi_v, sem),
        pltpu.async_copy(upd_hbm.at[pl.ds(us0 * S, 128)], uids_v.at[0], sem),
        pltpu.async_copy(upd_hbm.at[pl.ds(us0 * S + 128, 128)], uids_v.at[1], sem),
        pltpu.async_copy(embed_hbm.at[0], row0_v, sem),
        pltpu.async_copy(input_hbm.at[pl.ds(t0, TPW)], ids_v, sem),
        pltpu.async_copy(ttb_hbm.at[pl.ds(t0, TPW)], ttb_v, sem),
    ]
    for cp in stg:
        cp.wait()

    def tok_body(g, _):
        idv = ids_v[pl.ds(g * L, L)]
        ish = idv >= IVS
        tb = ttb_v[pl.ds(g * L, L)]
        slot = plsc.load_gather(hwpi_v, [tb])
        cell = slot * PLEN + (idv - IVS)
        row = g // 8
        col = (g % 8) * L
        bidx_v[row, pl.ds(col, L)] = jnp.where(ish, 0, idv)
        cells_v[row, pl.ds(col, L)] = jnp.where(ish, cell, 0)
        return 0
    lax.fori_loop(0, TPW // L, tok_body, 0)

    def cells_body(g, _):
        tb = utb_v[pl.ds(g * L, L)]
        slot = plsc.load_gather(hwpi_v, [tb])
        cell = slot * PLEN + uidx_v[pl.ds(g * L, L)]
        cells_all_v[pl.ds(g * L, L)] = cell
        return 0
    lax.fori_loop(0, U // L, cells_body, 0)

    for gl in range(2):
        npad = jnp.zeros((L,), jnp.int32)
        for ss in range(S):
            idsv = plsc.load_gather(
                uids_v, [jnp.full((L,), gl, jnp.int32), _iota() * S + ss])
            npad = npad + jnp.where(idsv == 0, 1, 0).astype(jnp.int32)
        denom = jnp.maximum(S - npad, 1).astype(jnp.float32)
        recip_v[pl.ds(gl * L, L)] = 1.0 / denom
        npadf_v[pl.ds(gl * L, L)] = npad.astype(jnp.float32)

    def init_body(i, _):
        initbuf_v[pl.ds(i * L, L)] = jnp.full((L,), -1, jnp.int32)
        return 0
    lax.fori_loop(0, 1024 // L, init_body, 0)
    cps_init = [pltpu.async_copy(initbuf_v,
                                 ver_s.at[pl.ds(s * 8192 + k * 1024, 1024)],
                                 sem)
                for k in range(8)]
    for cp in cps_init:
        cp.wait()
    plsc.subcore_barrier()

    for gl, gg in enumerate((s, 2 * NS - 1 - s)):
        myu = gg * L + _iota()
        mycell = cells_all_v[pl.ds(gg * L, L)]

        def win_body(cg, coll):
            for ol in range(L):
                ocell = _splat_ref(cells_all_v, cg * L + ol)
                ou = cg * L + ol
                coll = coll | ((mycell == ocell) & (ou > myu))
            return coll
        collided = lax.fori_loop(gg, U // L, win_body,
                                 jnp.zeros((L,), jnp.bool_))
        target = jnp.where(collided, NCELL + _iota(), mycell)
        scidx_v[gl, pl.ds(0, L)] = target
        scval_v[gl, pl.ds(0, L)] = myu
    for gl in range(2):
        pltpu.async_copy(scval_v.at[gl], ver_s.at[scidx_v.at[gl]], sem).wait()
    plsc.subcore_barrier()

    cps_ver = [pltpu.async_copy(ver_s.at[cells_v.at[j]], vvals_v.at[j], sem3)
               for j in range(4)]
    cps_base = [pltpu.async_copy(embed_hbm.at[bidx_v.at[j]],
                                 rows_v.at[pl.ds(j * 128, 128)], sem2)
                for j in range(4)]
    cp_ua = pltpu.async_copy(embed_hbm.at[uids_v.at[0]], urows_a, semu)

    for cp in cps_ver:
        cp.wait()

    def cmp_body(g, carry):
        cntb, cnte = carry
        idv = ids_v[pl.ds(g * L, L)]
        ish = idv >= IVS
        v = vvals_v[g // 8, pl.ds((g % 8) * L, L)]
        bm = ish & (v < 0)
        em = ish & (v >= 0)
        lt = g * L + _iota()
        bmi = jnp.where(bm, 1, 0).astype(jnp.int32)
        emi = jnp.where(em, 1, 0).astype(jnp.int32)
        db = jnp.maximum(cntb + lax.cumsum(bmi) - 1, 0)
        de = jnp.maximum(cnte + lax.cumsum(emi) - 1, 0)
        plsc.store_scatter(bufpos_v, [db // L, db % L], lt, mask=bm)
        plsc.store_scatter(encpos_v, [de // L, de % L], lt, mask=em)
        return (cntb + jnp.sum(bmi), cnte + jnp.sum(emi))
    cntb, cnte = lax.fori_loop(0, TPW // L, cmp_body,
                               (jnp.int32(0), jnp.int32(0)))

    def fill_body(g, carry):
        mb, me = carry
        lt = g * L + _iota()
        vb = jnp.where(lt < cntb, bufpos_v[g, pl.ds(0, L)], 0)
        vb = plsc.cummax(jnp.maximum(vb, mb))
        bufpos_v[g, pl.ds(0, L)] = vb
        ve = jnp.where(lt < cnte, encpos_v[g, pl.ds(0, L)], 0)
        ve = plsc.cummax(jnp.maximum(ve, me))
        encpos_v[g, pl.ds(0, L)] = ve
        return (plsc.load_gather(bufpos_v, [_bcast(g), _bcast(15)]),
                plsc.load_gather(encpos_v, [_bcast(g), _bcast(15)]))
    nb = (cntb + L - 1) // L
    ne = (cnte + L - 1) // L
    lax.fori_loop(0, jnp.maximum(nb, ne), fill_body,
                  (jnp.zeros((L,), jnp.int32), jnp.zeros((L,), jnp.int32)))

    for b in range(2):
        urows_v = urows_a
        if b == 0:
            cp_ua.wait()
        else:
            pltpu.async_copy(embed_hbm.at[uids_v.at[1]], urows_a, semu).wait()

        def enc_body(ul, _):
            u = b * L + ul
            rsp = _splat_ref(recip_v, u)
            nsp = _splat_ref(npadf_v, u)
            for cc in range(H // L):
                acc = urows_v[ul * S, pl.ds(cc * L, L)]
                for ss in range(1, S):
                    acc = acc + urows_v[ul * S + ss, pl.ds(cc * L, L)]
                acc = (acc - nsp * row0_v[pl.ds(cc * L, L)]) * rsp
                encbuf_v[u, pl.ds(cc * L, L)] = acc
            return 0
        lax.fori_loop(0, L, enc_body, 0)
    pltpu.sync_copy(encbuf_v, enc_s.at[pl.ds(us0, UPS)])


    cps_out = []
    for j in range(4):
        cps_base[j].wait()
        cps_out.append(pltpu.async_copy(rows_v.at[pl.ds(j * 128, 128)],
                                        out_hbm.at[pl.ds(t0 + j * 128, 128)],
                                        sem2))
    plsc.subcore_barrier()
    for cp in cps_out:
        cp.wait()

    def buf_body(g, _):
        pos = bufpos_v[g, pl.ds(0, L)]
        idv = plsc.load_gather(ids_v, [pos])
        tb = plsc.load_gather(ttb_v, [pos])
        slot = plsc.load_gather(hwpi_v, [tb])
        cell = slot * PLEN + (idv - IVS)
        pltpu.async_copy(pool_hbm.at[cell], hrows_v, sem).wait()
        outidx_v[0, pl.ds(0, L)] = t0 + pos
        pltpu.async_copy(hrows_v, out_hbm.at[outidx_v.at[0]], sem).wait()
        return 0
    lax.fori_loop(0, nb, buf_body, 0)

    def enc2_body(g, _):
        pos = encpos_v[g, pl.ds(0, L)]
        v = plsc.load_gather(vvals_v, [pos // 128, pos % 128])
        pltpu.async_copy(enc_s.at[v], hrows_v, sem).wait()
        outidx_v[0, pl.ds(0, L)] = t0 + pos
        pltpu.async_copy(hrows_v, out_hbm.at[outidx_v.at[0]], sem).wait()
        return 0
    lax.fori_loop(0, ne, enc2_body, 0)


@jax.jit
def kernel(input_, embed_weight, embedding_buffer, updates, updates_indices,
           update_to_batch, hyper_weight_pool_indices, token_to_batch_indices):
    pool = embedding_buffer.reshape(NCELL, H)
    upd = updates.reshape(U * S)
    run = pl.kernel(
        _body,
        out_type=jax.ShapeDtypeStruct((T, H), jnp.float32),
        mesh=plsc.VectorSubcoreMesh(core_axis_name="c", subcore_axis_name="s",
                                    num_cores=NC, num_subcores=NS),
        compiler_params=pltpu.CompilerParams(needs_layout_passes=False),
        scratch_types=[
            pltpu.VMEM((TPW,), jnp.int32),
            pltpu.VMEM((TPW,), jnp.int32),
            pltpu.VMEM((NPOOL,), jnp.int32),
            pltpu.VMEM((2, 128), jnp.int32),
            pltpu.VMEM((U,), jnp.int32),
            pltpu.VMEM((U,), jnp.int32),
            pltpu.VMEM((U,), jnp.int32),
            pltpu.VMEM((TPW, H), jnp.float32),
            pltpu.VMEM((L * S, H), jnp.float32),
            pltpu.VMEM((UPS, H), jnp.float32),
            pltpu.VMEM((H,), jnp.float32),
            pltpu.VMEM((UPS,), jnp.float32),
            pltpu.VMEM((UPS,), jnp.float32),
            pltpu.VMEM((1024,), jnp.int32),
            pltpu.VMEM((2, L), jnp.int32),
            pltpu.VMEM((2, L), jnp.int32),
            pltpu.VMEM((4, 128), jnp.int32),
            pltpu.VMEM((4, 128), jnp.int32),
            pltpu.VMEM((4, 128), jnp.int32),
            pltpu.VMEM((TPW // L, L), jnp.int32),
            pltpu.VMEM((TPW // L, L), jnp.int32),
            pltpu.VMEM((L, H), jnp.float32),
            pltpu.VMEM((1, L), jnp.int32),
            pltpu.VMEM_SHARED((U, H), jnp.float32),
            pltpu.VMEM_SHARED((NCELL + L,), jnp.int32),
            pltpu.SemaphoreType.DMA,
            pltpu.SemaphoreType.DMA,
            pltpu.SemaphoreType.DMA,
            pltpu.SemaphoreType.DMA,
        ],
    )
    return run(input_.astype(jnp.int32), embed_weight, pool, upd,
               updates_indices.astype(jnp.int32),
               update_to_batch.astype(jnp.int32),
               hyper_weight_pool_indices.astype(jnp.int32),
               token_to_batch_indices.astype(jnp.int32))


# speedup vs baseline: 1.0380x; 1.0380x over previous
import jax
import jax.numpy as jnp
from jax import lax
from jax.experimental import pallas as pl
from jax.experimental.pallas import tpu as pltpu
from jax.experimental.pallas import tpu_sc as plsc

IVS = 100000
T = 16384
H = 128
U = 512
S = 8
NPOOL = 64
PLEN = 2048
NCELL = NPOOL * PLEN
NC, NS = 2, 16
NW = NC * NS
TPW = T // NW
UPS = U // NS
L = 16


def _iota():
    return lax.iota(jnp.int32, L)


def _bcast(x):
    return jnp.broadcast_to(x, (L,)).astype(jnp.int32)


def _splat_ref(ref, idx):
    return plsc.load_gather(ref, [_bcast(idx)])


def _body(input_hbm, embed_hbm, pool_hbm, upd_hbm, uidx_hbm, utb_hbm,
          hwpi_hbm, ttb_hbm, out_hbm,
          ids_v, ttb_v, hwpi_v, uids_v, uidx_v, utb_v, cells_all_v,
          rows_v, urows_a, encbuf_v, row0_v, recip_v, npadf_v,
          initbuf_v, scidx_v, scval_v, bidx_v, cells_v, vvals_v,
          bufpos_v, encpos_v, hrows_v, outidx_v,
          enc_s, ver_s, sem, sem2, sem3, semu):
    c = lax.axis_index("c")
    s = lax.axis_index("s")
    wid = c * NS + s
    t0 = wid * TPW
    us0 = s * UPS

    stg = [
        pltpu.async_copy(uidx_hbm, uidx_v, sem),
        pltpu.async_copy(utb_hbm, utb_v, sem),
        pltpu.async_copy(hwpi_hbm, hwpi_v, sem),
        pltpu.async_copy(upd_hbm.at[pl.ds(us0 * S, 128)], uids_v.at[0], sem),
        pltpu.async_copy(upd_hbm.at[pl.ds(us0 * S + 128, 128)], uids_v.at[1], sem),
        pltpu.async_copy(embed_hbm.at[0], row0_v, sem),
        pltpu.async_copy(input_hbm.at[pl.ds(t0, TPW)], ids_v, sem),
        pltpu.async_copy(ttb_hbm.at[pl.ds(t0, TPW)], ttb_v, sem),
    ]
    for cp in stg:
        cp.wait()

    def tok_body(g, _):
        idv = ids_v[pl.ds(g * L, L)]
        ish = idv >= IVS
        tb = ttb_v[pl.ds(g * L, L)]
        slot = plsc.load_gather(hwpi_v, [tb])
        cell = slot * PLEN + (idv - IVS)
        row = g // 4
        col = (g % 4) * L
        bidx_v[row, pl.ds(col, L)] = jnp.where(ish, 0, idv)
        cells_v[row, pl.ds(col, L)] = jnp.where(ish, cell, 0)
        return 0
    lax.fori_loop(0, TPW // L, tok_body, 0)
    cps_base = [pltpu.async_copy(embed_hbm.at[bidx_v.at[j]],
                                 rows_v.at[pl.ds(j * 64, 64)], sem2)
                for j in range(8)]
    cp_ua = pltpu.async_copy(embed_hbm.at[uids_v.at[0]], urows_a, semu)

    def cells_body(g, _):
        tb = utb_v[pl.ds(g * L, L)]
        slot = plsc.load_gather(hwpi_v, [tb])
        cell = slot * PLEN + uidx_v[pl.ds(g * L, L)]
        cells_all_v[pl.ds(g * L, L)] = cell
        return 0
    lax.fori_loop(0, U // L, cells_body, 0)

    for gl in range(2):
        npad = jnp.zeros((L,), jnp.int32)
        for ss in range(S):
            idsv = plsc.load_gather(
                uids_v, [jnp.full((L,), gl, jnp.int32), _iota() * S + ss])
            npad = npad + jnp.where(idsv == 0, 1, 0).astype(jnp.int32)
        denom = jnp.maximum(S - npad, 1).astype(jnp.float32)
        recip_v[pl.ds(gl * L, L)] = 1.0 / denom
        npadf_v[pl.ds(gl * L, L)] = npad.astype(jnp.float32)

    def init_body(i, _):
        initbuf_v[pl.ds(i * L, L)] = jnp.full((L,), -1, jnp.int32)
        return 0
    lax.fori_loop(0, 1024 // L, init_body, 0)
    cps_init = [pltpu.async_copy(initbuf_v,
                                 ver_s.at[pl.ds(s * 8192 + k * 1024, 1024)],
                                 sem)
                for k in range(8)]
    for cp in cps_init:
        cp.wait()
    plsc.subcore_barrier()

    for gl, gg in enumerate((s, 2 * NS - 1 - s)):
        myu = gg * L + _iota()
        mycell = cells_all_v[pl.ds(gg * L, L)]

        def win_body(cg, coll):
            for ol in range(L):
                ocell = _splat_ref(cells_all_v, cg * L + ol)
                ou = cg * L + ol
                coll = coll | ((mycell == ocell) & (ou > myu))
            return coll
        collided = lax.fori_loop(gg, U // L, win_body,
                                 jnp.zeros((L,), jnp.bool_))
        target = jnp.where(collided, NCELL + _iota(), mycell)
        scidx_v[gl, pl.ds(0, L)] = target
        scval_v[gl, pl.ds(0, L)] = myu
    for gl in range(2):
        pltpu.async_copy(scval_v.at[gl], ver_s.at[scidx_v.at[gl]], sem).wait()
    plsc.subcore_barrier()

    cps_ver = [pltpu.async_copy(ver_s.at[cells_v.at[j]], vvals_v.at[j], sem3)
               for j in range(8)]

    for b in range(2):
        urows_v = urows_a
        if b == 0:
            cp_ua.wait()
        else:
            pltpu.async_copy(embed_hbm.at[uids_v.at[1]], urows_a, semu).wait()

        def enc_body(ul, _):
            u = b * L + ul
            rsp = _splat_ref(recip_v, u)
            nsp = _splat_ref(npadf_v, u)
            for cc in range(H // L):
                acc = urows_v[ul * S, pl.ds(cc * L, L)]
                for ss in range(1, S):
                    acc = acc + urows_v[ul * S + ss, pl.ds(cc * L, L)]
                acc = (acc - nsp * row0_v[pl.ds(cc * L, L)]) * rsp
                encbuf_v[u, pl.ds(cc * L, L)] = acc
            return 0
        lax.fori_loop(0, L, enc_body, 0)
    pltpu.sync_copy(encbuf_v, enc_s.at[pl.ds(us0, UPS)])

    for cp in cps_ver:
        cp.wait()

    def cmp_body(g, carry):
        cntb, cnte = carry
        idv = ids_v[pl.ds(g * L, L)]
        ish = idv >= IVS
        v = vvals_v[g // 4, pl.ds((g % 4) * L, L)]
        bm = ish & (v < 0)
        em = ish & (v >= 0)
        lt = g * L + _iota()
        bmi = jnp.where(bm, 1, 0).astype(jnp.int32)
        emi = jnp.where(em, 1, 0).astype(jnp.int32)
        db = jnp.maximum(cntb + lax.cumsum(bmi) - 1, 0)
        de = jnp.maximum(cnte + lax.cumsum(emi) - 1, 0)
        plsc.store_scatter(bufpos_v, [db // L, db % L], lt, mask=bm)
        plsc.store_scatter(encpos_v, [de // L, de % L], lt, mask=em)
        return (cntb + jnp.sum(bmi), cnte + jnp.sum(emi))
    cntb, cnte = lax.fori_loop(0, TPW // L, cmp_body,
                               (jnp.int32(0), jnp.int32(0)))

    def fill_body(g, carry):
        mb, me = carry
        lt = g * L + _iota()
        vb = jnp.where(lt < cntb, bufpos_v[g, pl.ds(0, L)], 0)
        vb = plsc.cummax(jnp.maximum(vb, mb))
        bufpos_v[g, pl.ds(0, L)] = vb
        ve = jnp.where(lt < cnte, encpos_v[g, pl.ds(0, L)], 0)
        ve = plsc.cummax(jnp.maximum(ve, me))
        encpos_v[g, pl.ds(0, L)] = ve
        return (plsc.load_gather(bufpos_v, [_bcast(g), _bcast(15)]),
                plsc.load_gather(encpos_v, [_bcast(g), _bcast(15)]))
    nb = (cntb + L - 1) // L
    ne = (cnte + L - 1) // L
    lax.fori_loop(0, jnp.maximum(nb, ne), fill_body,
                  (jnp.zeros((L,), jnp.int32), jnp.zeros((L,), jnp.int32)))

    cps_out = []
    for j in range(8):
        cps_base[j].wait()
        cps_out.append(pltpu.async_copy(rows_v.at[pl.ds(j * 64, 64)],
                                        out_hbm.at[pl.ds(t0 + j * 64, 64)],
                                        sem2))
    plsc.subcore_barrier()
    for cp in cps_out:
        cp.wait()

    def buf_body(g, _):
        pos = bufpos_v[g, pl.ds(0, L)]
        idv = plsc.load_gather(ids_v, [pos])
        tb = plsc.load_gather(ttb_v, [pos])
        slot = plsc.load_gather(hwpi_v, [tb])
        cell = slot * PLEN + (idv - IVS)
        pltpu.async_copy(pool_hbm.at[cell], hrows_v, sem).wait()
        outidx_v[0, pl.ds(0, L)] = t0 + pos
        pltpu.async_copy(hrows_v, out_hbm.at[outidx_v.at[0]], sem).wait()
        return 0
    lax.fori_loop(0, nb, buf_body, 0)

    def enc2_body(g, _):
        pos = encpos_v[g, pl.ds(0, L)]
        v = plsc.load_gather(vvals_v, [pos // 64, pos % 64])
        pltpu.async_copy(enc_s.at[v], hrows_v, sem).wait()
        outidx_v[0, pl.ds(0, L)] = t0 + pos
        pltpu.async_copy(hrows_v, out_hbm.at[outidx_v.at[0]], sem).wait()
        return 0
    lax.fori_loop(0, ne, enc2_body, 0)


@jax.jit
def kernel(input_, embed_weight, embedding_buffer, updates, updates_indices,
           update_to_batch, hyper_weight_pool_indices, token_to_batch_indices):
    pool = embedding_buffer.reshape(NCELL, H)
    upd = updates.reshape(U * S)
    run = pl.kernel(
        _body,
        out_type=jax.ShapeDtypeStruct((T, H), jnp.float32),
        mesh=plsc.VectorSubcoreMesh(core_axis_name="c", subcore_axis_name="s",
                                    num_cores=NC, num_subcores=NS),
        compiler_params=pltpu.CompilerParams(needs_layout_passes=False),
        scratch_types=[
            pltpu.VMEM((TPW,), jnp.int32),
            pltpu.VMEM((TPW,), jnp.int32),
            pltpu.VMEM((NPOOL,), jnp.int32),
            pltpu.VMEM((2, 128), jnp.int32),
            pltpu.VMEM((U,), jnp.int32),
            pltpu.VMEM((U,), jnp.int32),
            pltpu.VMEM((U,), jnp.int32),
            pltpu.VMEM((TPW, H), jnp.float32),
            pltpu.VMEM((L * S, H), jnp.float32),
            pltpu.VMEM((UPS, H), jnp.float32),
            pltpu.VMEM((H,), jnp.float32),
            pltpu.VMEM((UPS,), jnp.float32),
            pltpu.VMEM((UPS,), jnp.float32),
            pltpu.VMEM((1024,), jnp.int32),
            pltpu.VMEM((2, L), jnp.int32),
            pltpu.VMEM((2, L), jnp.int32),
            pltpu.VMEM((8, 64), jnp.int32),
            pltpu.VMEM((8, 64), jnp.int32),
            pltpu.VMEM((8, 64), jnp.int32),
            pltpu.VMEM((TPW // L, L), jnp.int32),
            pltpu.VMEM((TPW // L, L), jnp.int32),
            pltpu.VMEM((L, H), jnp.float32),
            pltpu.VMEM((1, L), jnp.int32),
            pltpu.VMEM_SHARED((U, H), jnp.float32),
            pltpu.VMEM_SHARED((NCELL + L,), jnp.int32),
            pltpu.SemaphoreType.DMA,
            pltpu.SemaphoreType.DMA,
            pltpu.SemaphoreType.DMA,
            pltpu.SemaphoreType.DMA,
        ],
    )
    return run(input_.astype(jnp.int32), embed_weight, pool, upd,
               updates_indices.astype(jnp.int32),
               update_to_batch.astype(jnp.int32),
               hyper_weight_pool_indices.astype(jnp.int32),
               token_to_batch_indices.astype(jnp.int32))


# speedup vs baseline: 1.0398x; 1.0017x over previous
import jax
import jax.numpy as jnp
from jax import lax
from jax.experimental import pallas as pl
from jax.experimental.pallas import tpu as pltpu
from jax.experimental.pallas import tpu_sc as plsc

IVS = 100000
T = 16384
H = 128
U = 512
S = 8
NPOOL = 64
PLEN = 2048
NCELL = NPOOL * PLEN
NC, NS = 2, 16
NW = NC * NS
TPW = T // NW
UPS = U // NS
L = 16


def _iota():
    return lax.iota(jnp.int32, L)


def _bcast(x):
    return jnp.broadcast_to(x, (L,)).astype(jnp.int32)


def _splat_ref(ref, idx):
    return plsc.load_gather(ref, [_bcast(idx)])


def _body(input_hbm, embed_hbm, pool_hbm, upd_hbm, uidx_hbm, utb_hbm,
          hwpi_hbm, ttb_hbm, out_hbm,
          ids_v, ttb_v, hwpi_v, uids_v, uidx_v, utb_v, cells_all_v,
          rows_v, urows_a, encbuf_v, row0_v, recip_v, npadf_v,
          initbuf_v, scidx_v, scval_v, bidx_v, cells_v, vvals_v,
          bufpos_v, encpos_v, hrows_v, outidx_v,
          enc_s, ver_s, sem, sem2, sem3, semu):
    c = lax.axis_index("c")
    s = lax.axis_index("s")
    wid = c * NS + s
    t0 = wid * TPW
    us0 = s * UPS

    stg = [
        pltpu.async_copy(uidx_hbm, uidx_v, sem),
        pltpu.async_copy(utb_hbm, utb_v, sem),
        pltpu.async_copy(hwpi_hbm, hwpi_v, sem),
        pltpu.async_copy(upd_hbm.at[pl.ds(us0 * S, 128)], uids_v.at[0], sem),
        pltpu.async_copy(upd_hbm.at[pl.ds(us0 * S + 128, 128)], uids_v.at[1], sem),
        pltpu.async_copy(embed_hbm.at[0], row0_v, sem),
        pltpu.async_copy(input_hbm.at[pl.ds(t0, TPW)], ids_v, sem),
        pltpu.async_copy(ttb_hbm.at[pl.ds(t0, TPW)], ttb_v, sem),
    ]
    for cp in stg:
        cp.wait()

    def tok_body(g, _):
        idv = ids_v[pl.ds(g * L, L)]
        ish = idv >= IVS
        tb = ttb_v[pl.ds(g * L, L)]
        slot = plsc.load_gather(hwpi_v, [tb])
        cell = slot * PLEN + (idv - IVS)
        row = g // 4
        col = (g % 4) * L
        bidx_v[row, pl.ds(col, L)] = jnp.where(ish, 0, idv)
        cells_v[row, pl.ds(col, L)] = jnp.where(ish, cell, 0)
        return 0
    lax.fori_loop(0, TPW // L, tok_body, 0)
    cps_base = [pltpu.async_copy(embed_hbm.at[bidx_v.at[j]],
                                 rows_v.at[pl.ds(j * 64, 64)], sem2,
                                 priority=1)
                for j in range(8)]
    cp_ua = pltpu.async_copy(embed_hbm.at[uids_v.at[0]], urows_a, semu)

    def cells_body(g, _):
        tb = utb_v[pl.ds(g * L, L)]
        slot = plsc.load_gather(hwpi_v, [tb])
        cell = slot * PLEN + uidx_v[pl.ds(g * L, L)]
        cells_all_v[pl.ds(g * L, L)] = cell
        return 0
    lax.fori_loop(0, U // L, cells_body, 0)

    for gl in range(2):
        npad = jnp.zeros((L,), jnp.int32)
        for ss in range(S):
            idsv = plsc.load_gather(
                uids_v, [jnp.full((L,), gl, jnp.int32), _iota() * S + ss])
            npad = npad + jnp.where(idsv == 0, 1, 0).astype(jnp.int32)
        denom = jnp.maximum(S - npad, 1).astype(jnp.float32)
        recip_v[pl.ds(gl * L, L)] = 1.0 / denom
        npadf_v[pl.ds(gl * L, L)] = npad.astype(jnp.float32)

    def init_body(i, _):
        initbuf_v[pl.ds(i * L, L)] = jnp.full((L,), -1, jnp.int32)
        return 0
    lax.fori_loop(0, 1024 // L, init_body, 0)
    cps_init = [pltpu.async_copy(initbuf_v,
                                 ver_s.at[pl.ds(s * 8192 + k * 1024, 1024)],
                                 sem)
                for k in range(8)]
    for cp in cps_init:
        cp.wait()
    plsc.subcore_barrier()

    for gl, gg in enumerate((s, 2 * NS - 1 - s)):
        myu = gg * L + _iota()
        mycell = cells_all_v[pl.ds(gg * L, L)]

        def win_body(cg, coll):
            for ol in range(L):
                ocell = _splat_ref(cells_all_v, cg * L + ol)
                ou = cg * L + ol
                coll = coll | ((mycell == ocell) & (ou > myu))
            return coll
        collided = lax.fori_loop(gg, U // L, win_body,
                                 jnp.zeros((L,), jnp.bool_))
        target = jnp.where(collided, NCELL + _iota(), mycell)
        scidx_v[gl, pl.ds(0, L)] = target
        scval_v[gl, pl.ds(0, L)] = myu
    for gl in range(2):
        pltpu.async_copy(scval_v.at[gl], ver_s.at[scidx_v.at[gl]], sem).wait()
    plsc.subcore_barrier()

    cps_ver = [pltpu.async_copy(ver_s.at[cells_v.at[j]], vvals_v.at[j], sem3)
               for j in range(8)]

    for b in range(2):
        urows_v = urows_a
        if b == 0:
            cp_ua.wait()
        else:
            pltpu.async_copy(embed_hbm.at[uids_v.at[1]], urows_a, semu).wait()

        def enc_body(ul, _):
            u = b * L + ul
            rsp = _splat_ref(recip_v, u)
            nsp = _splat_ref(npadf_v, u)
            for cc in range(H // L):
                acc = urows_v[ul * S, pl.ds(cc * L, L)]
                for ss in range(1, S):
                    acc = acc + urows_v[ul * S + ss, pl.ds(cc * L, L)]
                acc = (acc - nsp * row0_v[pl.ds(cc * L, L)]) * rsp
                encbuf_v[u, pl.ds(cc * L, L)] = acc
            return 0
        lax.fori_loop(0, L, enc_body, 0)
    pltpu.sync_copy(encbuf_v, enc_s.at[pl.ds(us0, UPS)])

    for cp in cps_ver:
        cp.wait()

    def cmp_body(g, carry):
        cntb, cnte = carry
        idv = ids_v[pl.ds(g * L, L)]
        ish = idv >= IVS
        v = vvals_v[g // 4, pl.ds((g % 4) * L, L)]
        bm = ish & (v < 0)
        em = ish & (v >= 0)
        lt = g * L + _iota()
        bmi = jnp.where(bm, 1, 0).astype(jnp.int32)
        emi = jnp.where(em, 1, 0).astype(jnp.int32)
        db = jnp.maximum(cntb + lax.cumsum(bmi) - 1, 0)
        de = jnp.maximum(cnte + lax.cumsum(emi) - 1, 0)
        plsc.store_scatter(bufpos_v, [db // L, db % L], lt, mask=bm)
        plsc.store_scatter(encpos_v, [de // L, de % L], lt, mask=em)
        return (cntb + jnp.sum(bmi), cnte + jnp.sum(emi))
    cntb, cnte = lax.fori_loop(0, TPW // L, cmp_body,
                               (jnp.int32(0), jnp.int32(0)))

    def fill_body(g, carry):
        mb, me = carry
        lt = g * L + _iota()
        vb = jnp.where(lt < cntb, bufpos_v[g, pl.ds(0, L)], 0)
        vb = plsc.cummax(jnp.maximum(vb, mb))
        bufpos_v[g, pl.ds(0, L)] = vb
        ve = jnp.where(lt < cnte, encpos_v[g, pl.ds(0, L)], 0)
        ve = plsc.cummax(jnp.maximum(ve, me))
        encpos_v[g, pl.ds(0, L)] = ve
        return (plsc.load_gather(bufpos_v, [_bcast(g), _bcast(15)]),
                plsc.load_gather(encpos_v, [_bcast(g), _bcast(15)]))
    nb = (cntb + L - 1) // L
    ne = (cnte + L - 1) // L
    lax.fori_loop(0, jnp.maximum(nb, ne), fill_body,
                  (jnp.zeros((L,), jnp.int32), jnp.zeros((L,), jnp.int32)))

    cps_out = []
    for j in range(8):
        cps_base[j].wait()
        cps_out.append(pltpu.async_copy(rows_v.at[pl.ds(j * 64, 64)],
                                        out_hbm.at[pl.ds(t0 + j * 64, 64)],
                                        sem2))
    plsc.subcore_barrier()
    for cp in cps_out:
        cp.wait()

    def buf_body(g, _):
        pos = bufpos_v[g, pl.ds(0, L)]
        idv = plsc.load_gather(ids_v, [pos])
        tb = plsc.load_gather(ttb_v, [pos])
        slot = plsc.load_gather(hwpi_v, [tb])
        cell = slot * PLEN + (idv - IVS)
        pltpu.async_copy(pool_hbm.at[cell], hrows_v, sem).wait()
        outidx_v[0, pl.ds(0, L)] = t0 + pos
        pltpu.async_copy(hrows_v, out_hbm.at[outidx_v.at[0]], sem).wait()
        return 0
    lax.fori_loop(0, nb, buf_body, 0)

    def enc2_body(g, _):
        pos = encpos_v[g, pl.ds(0, L)]
        v = plsc.load_gather(vvals_v, [pos // 64, pos % 64])
        pltpu.async_copy(enc_s.at[v], hrows_v, sem).wait()
        outidx_v[0, pl.ds(0, L)] = t0 + pos
        pltpu.async_copy(hrows_v, out_hbm.at[outidx_v.at[0]], sem).wait()
        return 0
    lax.fori_loop(0, ne, enc2_body, 0)


@jax.jit
def kernel(input_, embed_weight, embedding_buffer, updates, updates_indices,
           update_to_batch, hyper_weight_pool_indices, token_to_batch_indices):
    pool = embedding_buffer.reshape(NCELL, H)
    upd = updates.reshape(U * S)
    run = pl.kernel(
        _body,
        out_type=jax.ShapeDtypeStruct((T, H), jnp.float32),
        mesh=plsc.VectorSubcoreMesh(core_axis_name="c", subcore_axis_name="s",
                                    num_cores=NC, num_subcores=NS),
        compiler_params=pltpu.CompilerParams(needs_layout_passes=False),
        scratch_types=[
            pltpu.VMEM((TPW,), jnp.int32),
            pltpu.VMEM((TPW,), jnp.int32),
            pltpu.VMEM((NPOOL,), jnp.int32),
            pltpu.VMEM((2, 128), jnp.int32),
            pltpu.VMEM((U,), jnp.int32),
            pltpu.VMEM((U,), jnp.int32),
            pltpu.VMEM((U,), jnp.int32),
            pltpu.VMEM((TPW, H), jnp.float32),
            pltpu.VMEM((L * S, H), jnp.float32),
            pltpu.VMEM((UPS, H), jnp.float32),
            pltpu.VMEM((H,), jnp.float32),
            pltpu.VMEM((UPS,), jnp.float32),
            pltpu.VMEM((UPS,), jnp.float32),
            pltpu.VMEM((1024,), jnp.int32),
            pltpu.VMEM((2, L), jnp.int32),
            pltpu.VMEM((2, L), jnp.int32),
            pltpu.VMEM((8, 64), jnp.int32),
            pltpu.VMEM((8, 64), jnp.int32),
            pltpu.VMEM((8, 64), jnp.int32),
            pltpu.VMEM((TPW // L, L), jnp.int32),
            pltpu.VMEM((TPW // L, L), jnp.int32),
            pltpu.VMEM((L, H), jnp.float32),
            pltpu.VMEM((1, L), jnp.int32),
            pltpu.VMEM_SHARED((U, H), jnp.float32),
            pltpu.VMEM_SHARED((NCELL + L,), jnp.int32),
            pltpu.SemaphoreType.DMA,
            pltpu.SemaphoreType.DMA,
            pltpu.SemaphoreType.DMA,
            pltpu.SemaphoreType.DMA,
        ],
    )
    return run(input_.astype(jnp.int32), embed_weight, pool, upd,
               updates_indices.astype(jnp.int32),
               update_to_batch.astype(jnp.int32),
               hyper_weight_pool_indices.astype(jnp.int32),
               token_to_batch_indices.astype(jnp.int32))


# speedup vs baseline: 1.0485x; 1.0083x over previous
import jax
import jax.numpy as jnp
from jax import lax
from jax.experimental import pallas as pl
from jax.experimental.pallas import tpu as pltpu
from jax.experimental.pallas import tpu_sc as plsc

IVS = 100000
T = 16384
H = 128
U = 512
S = 8
NPOOL = 64
PLEN = 2048
NCELL = NPOOL * PLEN
NC, NS = 2, 16
NW = NC * NS
TPW = T // NW
UPS = U // NS
L = 16


def _iota():
    return lax.iota(jnp.int32, L)


def _bcast(x):
    return jnp.broadcast_to(x, (L,)).astype(jnp.int32)


def _splat_ref(ref, idx):
    return plsc.load_gather(ref, [_bcast(idx)])


def _body(input_hbm, embed_hbm, pool_hbm, upd_hbm, uidx_hbm, utb_hbm,
          hwpi_hbm, ttb_hbm, out_hbm,
          ids_v, ttb_v, hwpi_v, uids_v, uidx_v, utb_v, cells_all_v,
          rows_v, urows_a, encbuf_v, row0_v, recip_v, npadf_v,
          initbuf_v, scidx_v, scval_v, bidx_v, cells_v, vvals_v,
          bufpos_v, encpos_v, hrows_v, outidx_v,
          enc_s, ver_s, sem, sem2, sem3, semu):
    c = lax.axis_index("c")
    s = lax.axis_index("s")
    wid = c * NS + s
    t0 = wid * TPW
    us0 = s * UPS

    stg = [
        pltpu.async_copy(uidx_hbm, uidx_v, sem),
        pltpu.async_copy(utb_hbm, utb_v, sem),
        pltpu.async_copy(hwpi_hbm, hwpi_v, sem),
        pltpu.async_copy(upd_hbm.at[pl.ds(us0 * S, 128)], uids_v.at[0], sem),
        pltpu.async_copy(upd_hbm.at[pl.ds(us0 * S + 128, 128)], uids_v.at[1], sem),
        pltpu.async_copy(embed_hbm.at[0], row0_v, sem),
        pltpu.async_copy(input_hbm.at[pl.ds(t0, TPW)], ids_v, sem),
        pltpu.async_copy(ttb_hbm.at[pl.ds(t0, TPW)], ttb_v, sem),
    ]
    for cp in stg:
        cp.wait()

    def tok_body(g, _):
        idv = ids_v[pl.ds(g * L, L)]
        ish = idv >= IVS
        tb = ttb_v[pl.ds(g * L, L)]
        slot = plsc.load_gather(hwpi_v, [tb])
        cell = slot * PLEN + (idv - IVS)
        row = g // 4
        col = (g % 4) * L
        bidx_v[row, pl.ds(col, L)] = jnp.where(ish, 0, idv)
        cells_v[row, pl.ds(col, L)] = jnp.where(ish, cell, 0)
        return 0
    lax.fori_loop(0, TPW // L, tok_body, 0)
    cps_base = [pltpu.async_copy(embed_hbm.at[bidx_v.at[j]],
                                 rows_v.at[pl.ds(j * 64, 64)], sem2)
                for j in range(8)]
    cp_ua = pltpu.async_copy(embed_hbm.at[uids_v.at[0]], urows_a, semu)

    def cells_body(g, _):
        tb = utb_v[pl.ds(g * L, L)]
        slot = plsc.load_gather(hwpi_v, [tb])
        cell = slot * PLEN + uidx_v[pl.ds(g * L, L)]
        cells_all_v[pl.ds(g * L, L)] = cell
        return 0
    lax.fori_loop(0, U // L, cells_body, 0)

    for gl in range(2):
        npad = jnp.zeros((L,), jnp.int32)
        for ss in range(S):
            idsv = plsc.load_gather(
                uids_v, [jnp.full((L,), gl, jnp.int32), _iota() * S + ss])
            npad = npad + jnp.where(idsv == 0, 1, 0).astype(jnp.int32)
        denom = jnp.maximum(S - npad, 1).astype(jnp.float32)
        recip_v[pl.ds(gl * L, L)] = 1.0 / denom
        npadf_v[pl.ds(gl * L, L)] = npad.astype(jnp.float32)

    def init_body(i, _):
        initbuf_v[pl.ds(i * L, L)] = jnp.full((L,), -1, jnp.int32)
        return 0
    lax.fori_loop(0, 1024 // L, init_body, 0)
    cps_init = [pltpu.async_copy(initbuf_v,
                                 ver_s.at[pl.ds(s * 8192 + k * 1024, 1024)],
                                 sem)
                for k in range(8)]
    for cp in cps_init:
        cp.wait()
    plsc.subcore_barrier()

    for gl, gg in enumerate((s, 2 * NS - 1 - s)):
        myu = gg * L + _iota()
        mycell = cells_all_v[pl.ds(gg * L, L)]

        def win_body(cg, coll):
            for ol in range(L):
                ocell = _splat_ref(cells_all_v, cg * L + ol)
                ou = cg * L + ol
                coll = coll | ((mycell == ocell) & (ou > myu))
            return coll
        collided = lax.fori_loop(gg, U // L, win_body,
                                 jnp.zeros((L,), jnp.bool_))
        target = jnp.where(collided, NCELL + _iota(), mycell)
        scidx_v[gl, pl.ds(0, L)] = target
        scval_v[gl, pl.ds(0, L)] = myu
    for gl in range(2):
        pltpu.async_copy(scval_v.at[gl], ver_s.at[scidx_v.at[gl]], sem).wait()
    plsc.subcore_barrier()

    cps_ver = [pltpu.async_copy(ver_s.at[cells_v.at[j]], vvals_v.at[j], sem3)
               for j in range(8)]

    for b in range(2):
        urows_v = urows_a
        if b == 0:
            cp_ua.wait()
        else:
            pltpu.async_copy(embed_hbm.at[uids_v.at[1]], urows_a, semu).wait()

        def enc_body(ul, _):
            u = b * L + ul
            rsp = _splat_ref(recip_v, u)
            nsp = _splat_ref(npadf_v, u)
            for cc in range(H // L):
                acc = urows_v[ul * S, pl.ds(cc * L, L)]
                for ss in range(1, S):
                    acc = acc + urows_v[ul * S + ss, pl.ds(cc * L, L)]
                acc = (acc - nsp * row0_v[pl.ds(cc * L, L)]) * rsp
                encbuf_v[u, pl.ds(cc * L, L)] = acc
            return 0
        lax.fori_loop(0, L, enc_body, 0)
    pltpu.sync_copy(encbuf_v, enc_s.at[pl.ds(us0, UPS)])

    for cp in cps_ver:
        cp.wait()

    def cmp_body(g, carry):
        cntb, cnte = carry
        idv = ids_v[pl.ds(g * L, L)]
        ish = idv >= IVS
        v = vvals_v[g // 4, pl.ds((g % 4) * L, L)]
        bm = ish & (v < 0)
        em = ish & (v >= 0)
        lt = g * L + _iota()
        bmi = jnp.where(bm, 1, 0).astype(jnp.int32)
        emi = jnp.where(em, 1, 0).astype(jnp.int32)
        db = jnp.maximum(cntb + lax.cumsum(bmi) - 1, 0)
        de = jnp.maximum(cnte + lax.cumsum(emi) - 1, 0)
        plsc.store_scatter(bufpos_v, [db // L, db % L], lt, mask=bm)
        plsc.store_scatter(encpos_v, [de // L, de % L], lt, mask=em)
        return (cntb + jnp.sum(bmi), cnte + jnp.sum(emi))
    cntb, cnte = lax.fori_loop(0, TPW // L, cmp_body,
                               (jnp.int32(0), jnp.int32(0)))

    def fill_body(g, carry):
        mb, me = carry
        lt = g * L + _iota()
        vb = jnp.where(lt < cntb, bufpos_v[g, pl.ds(0, L)], 0)
        vb = plsc.cummax(jnp.maximum(vb, mb))
        bufpos_v[g, pl.ds(0, L)] = vb
        ve = jnp.where(lt < cnte, encpos_v[g, pl.ds(0, L)], 0)
        ve = plsc.cummax(jnp.maximum(ve, me))
        encpos_v[g, pl.ds(0, L)] = ve
        return (plsc.load_gather(bufpos_v, [_bcast(g), _bcast(15)]),
                plsc.load_gather(encpos_v, [_bcast(g), _bcast(15)]))
    nb = (cntb + L - 1) // L
    ne = (cnte + L - 1) // L
    lax.fori_loop(0, jnp.maximum(nb, ne), fill_body,
                  (jnp.zeros((L,), jnp.int32), jnp.zeros((L,), jnp.int32)))

    cps_out = []
    for j in range(8):
        cps_base[j].wait()
        cps_out.append(pltpu.async_copy(rows_v.at[pl.ds(j * 64, 64)],
                                        out_hbm.at[pl.ds(t0 + j * 64, 64)],
                                        sem2))
    plsc.subcore_barrier()
    for cp in cps_out:
        cp.wait()

    def buf_body(g, _):
        pos = bufpos_v[g, pl.ds(0, L)]
        idv = plsc.load_gather(ids_v, [pos])
        tb = plsc.load_gather(ttb_v, [pos])
        slot = plsc.load_gather(hwpi_v, [tb])
        cell = slot * PLEN + (idv - IVS)
        pltpu.async_copy(pool_hbm.at[cell], hrows_v, sem).wait()
        outidx_v[0, pl.ds(0, L)] = t0 + pos
        pltpu.async_copy(hrows_v, out_hbm.at[outidx_v.at[0]], sem).wait()
        return 0
    lax.fori_loop(0, nb, buf_body, 0)

    def enc2_body(g, _):
        pos = encpos_v[g, pl.ds(0, L)]
        v = plsc.load_gather(vvals_v, [pos // 64, pos % 64])
        pltpu.async_copy(enc_s.at[v], hrows_v, sem).wait()
        outidx_v[0, pl.ds(0, L)] = t0 + pos
        pltpu.async_copy(hrows_v, out_hbm.at[outidx_v.at[0]], sem).wait()
        return 0
    lax.fori_loop(0, ne, enc2_body, 0)


@jax.jit
def kernel(input_, embed_weight, embedding_buffer, updates, updates_indices,
           update_to_batch, hyper_weight_pool_indices, token_to_batch_indices):
    pool = embedding_buffer.reshape(NCELL, H)
    upd = updates.reshape(U * S)
    run = pl.kernel(
        _body,
        out_type=jax.ShapeDtypeStruct((T, H), jnp.float32),
        mesh=plsc.VectorSubcoreMesh(core_axis_name="c", subcore_axis_name="s",
                                    num_cores=NC, num_subcores=NS),
        compiler_params=pltpu.CompilerParams(needs_layout_passes=False),
        scratch_types=[
            pltpu.VMEM((TPW,), jnp.int32),
            pltpu.VMEM((TPW,), jnp.int32),
            pltpu.VMEM((NPOOL,), jnp.int32),
            pltpu.VMEM((2, 128), jnp.int32),
            pltpu.VMEM((U,), jnp.int32),
            pltpu.VMEM((U,), jnp.int32),
            pltpu.VMEM((U,), jnp.int32),
            pltpu.VMEM((TPW, H), jnp.float32),
            pltpu.VMEM((L * S, H), jnp.float32),
            pltpu.VMEM((UPS, H), jnp.float32),
            pltpu.VMEM((H,), jnp.float32),
            pltpu.VMEM((UPS,), jnp.float32),
            pltpu.VMEM((UPS,), jnp.float32),
            pltpu.VMEM((1024,), jnp.int32),
            pltpu.VMEM((2, L), jnp.int32),
            pltpu.VMEM((2, L), jnp.int32),
            pltpu.VMEM((8, 64), jnp.int32),
            pltpu.VMEM((8, 64), jnp.int32),
            pltpu.VMEM((8, 64), jnp.int32),
            pltpu.VMEM((TPW // L, L), jnp.int32),
            pltpu.VMEM((TPW // L, L), jnp.int32),
            pltpu.VMEM((L, H), jnp.float32),
            pltpu.VMEM((1, L), jnp.int32),
            pltpu.VMEM_SHARED((U, H), jnp.float32),
            pltpu.VMEM_SHARED((NCELL + L,), jnp.int32),
            pltpu.SemaphoreType.DMA,
            pltpu.SemaphoreType.DMA,
            pltpu.SemaphoreType.DMA,
            pltpu.SemaphoreType.DMA,
        ],
    )
    return run(input_.astype(jnp.int32), embed_weight, pool, upd,
               updates_indices.astype(jnp.int32),
               update_to_batch.astype(jnp.int32),
               hyper_weight_pool_indices.astype(jnp.int32),
               token_to_batch_indices.astype(jnp.int32))
